# Initial kernel scaffold; baseline (speedup 1.0000x reference)
import jax
import jax.numpy as jnp
from jax.experimental import pallas as pl


def kernel(pred_logits, pred_boxes, tgt_labels, tgt_boxes):
    raise NotImplementedError("write your pallas kernel here")


# speedup vs baseline: 8.7284x; 8.7284x over previous
import functools

import jax
import jax.numpy as jnp
from jax import lax
from jax.experimental import pallas as pl
from jax.experimental.pallas import tpu as pltpu
from jax.experimental.pallas import tpu_sc as plsc

COST_CLASS, COST_BBOX, COST_GIOU = 2.0, 5.0, 2.0
FOCAL_ALPHA = 0.25

BS, NQ, NC, NTGT = 16, 900, 91, 50
LPAD = 64
NW = 32
QH = NQ // 2
ROWS_WORDS = QH * NC
ROWS_BUF = 40960
TOTAL_WORDS = BS * NQ * NC
OUT_WORDS = QH * LPAD


def _sc_gather_body(logits_hbm, labels_hbm, out_hbm, rows_v, out_v, lab_v):
    wid = lax.axis_index("s") * 2 + lax.axis_index("c")
    b = wid // 2
    pltpu.sync_copy(labels_hbm.at[pl.ds(b * LPAD, LPAD)], lab_v)

    start = wid * ROWS_WORDS
    aligned = jnp.minimum((start // 8) * 8, TOTAL_WORDS - ROWS_BUF)
    delta = start - aligned
    pltpu.sync_copy(logits_hbm.at[pl.ds(aligned, ROWS_BUF)], rows_v)

    labs = [lab_v[pl.ds(c4 * 16, 16)] for c4 in range(4)]

    def rowfn(i, base):
        for c4 in range(4):
            idx = base + labs[c4]
            out_v[pl.ds(i * LPAD + c4 * 16, 16)] = plsc.load_gather(rows_v, [idx])
        return base + NC

    lax.fori_loop(0, QH, rowfn, delta)
    pltpu.sync_copy(out_v, out_hbm.at[pl.ds(wid * OUT_WORDS, OUT_WORDS)])


@functools.partial(jax.jit, static_argnums=())
def _sc_gather(logits_flat, labels_flat):
    run = pl.kernel(
        _sc_gather_body,
        out_type=jax.ShapeDtypeStruct((NW * OUT_WORDS,), jnp.float32),
        mesh=plsc.VectorSubcoreMesh(core_axis_name="c", subcore_axis_name="s"),
        scratch_types=[
            pltpu.VMEM((ROWS_BUF,), jnp.float32),
            pltpu.VMEM((OUT_WORDS,), jnp.float32),
            pltpu.VMEM((LPAD,), jnp.int32),
        ],
        compiler_params=pltpu.CompilerParams(needs_layout_passes=False),
    )
    return run(logits_flat, labels_flat)


def _matcher_body(g_ref, pb_ref, tb_ref, idxi_ref, idxj_ref):
    nq = g_ref.shape[1]
    ntgt = NTGT

    g = g_ref[0][:, :ntgt]
    p = jax.nn.sigmoid(g)
    neg_cost = (1.0 - FOCAL_ALPHA) * (p * p) * (-jnp.log(1.0 - p + 1e-8))
    pos_cost = FOCAL_ALPHA * ((1.0 - p) * (1.0 - p)) * (-jnp.log(p + 1e-8))
    cost_class = pos_cost - neg_cost

    ocx = pb_ref[0, 0, :].reshape(nq, 1)
    ocy = pb_ref[0, 1, :].reshape(nq, 1)
    ow = pb_ref[0, 2, :].reshape(nq, 1)
    oh = pb_ref[0, 3, :].reshape(nq, 1)
    tcx = tb_ref[0, 0, :].reshape(1, ntgt)
    tcy = tb_ref[0, 1, :].reshape(1, ntgt)
    tw = tb_ref[0, 2, :].reshape(1, ntgt)
    th = tb_ref[0, 3, :].reshape(1, ntgt)

    cost_bbox = (jnp.abs(ocx - tcx) + jnp.abs(ocy - tcy)
                 + jnp.abs(ow - tw) + jnp.abs(oh - th))

    ox1, oy1 = ocx - 0.5 * ow, ocy - 0.5 * oh
    ox2, oy2 = ocx + 0.5 * ow, ocy + 0.5 * oh
    tx1, ty1 = tcx - 0.5 * tw, tcy - 0.5 * th
    tx2, ty2 = tcx + 0.5 * tw, tcy + 0.5 * th

    area1 = (ox2 - ox1) * (oy2 - oy1)
    area2 = (tx2 - tx1) * (ty2 - ty1)
    wx = jnp.maximum(jnp.minimum(ox2, tx2) - jnp.maximum(ox1, tx1), 0.0)
    wy = jnp.maximum(jnp.minimum(oy2, ty2) - jnp.maximum(oy1, ty1), 0.0)
    inter = wx * wy
    union = area1 + area2 - inter
    iou = inter / (union + 1e-9)
    w2x = jnp.maximum(jnp.maximum(ox2, tx2) - jnp.minimum(ox1, tx1), 0.0)
    w2y = jnp.maximum(jnp.maximum(oy2, ty2) - jnp.minimum(oy1, ty1), 0.0)
    area = w2x * w2y
    giou = iou - (area - union) / (area + 1e-9)

    C = COST_BBOX * cost_bbox + COST_CLASS * cost_class + COST_GIOU * (-giou)

    mn = jnp.min(C, axis=0)
    qio = lax.broadcasted_iota(jnp.int32, (nq, ntgt), 0)
    idx = jnp.min(jnp.where(C == mn[None, :], qio, nq), axis=0)
    idxi_ref[0] = idx.reshape(1, ntgt).astype(jnp.int32)
    idxj_ref[0] = lax.broadcasted_iota(jnp.int32, (1, ntgt), 1)


def kernel(pred_logits, pred_boxes, tgt_labels, tgt_boxes):
    bs, nq, nc = pred_logits.shape
    ntgt = tgt_labels.shape[1]
    labels_pad = jnp.pad(tgt_labels, ((0, 0), (0, LPAD - ntgt))).reshape(-1)
    gathered = _sc_gather(pred_logits.reshape(-1), labels_pad)
    g3 = gathered.reshape(bs, nq, LPAD)

    pb = pred_boxes.transpose(0, 2, 1)
    tb = tgt_boxes.transpose(0, 2, 1)

    idxi, idxj = pl.pallas_call(
        _matcher_body,
        grid=(bs,),
        in_specs=[
            pl.BlockSpec((1, nq, LPAD), lambda b: (b, 0, 0)),
            pl.BlockSpec((1, 4, nq), lambda b: (b, 0, 0)),
            pl.BlockSpec((1, 4, ntgt), lambda b: (b, 0, 0)),
        ],
        out_specs=(
            pl.BlockSpec((1, 1, ntgt), lambda b: (b, 0, 0)),
            pl.BlockSpec((1, 1, ntgt), lambda b: (b, 0, 0)),
        ),
        out_shape=(
            jax.ShapeDtypeStruct((bs, 1, ntgt), jnp.int32),
            jax.ShapeDtypeStruct((bs, 1, ntgt), jnp.int32),
        ),
    )(g3, pb, tb)
    return idxi.reshape(bs, ntgt), idxj.reshape(bs, ntgt)


# speedup vs baseline: 17.4300x; 1.9969x over previous
import jax
import jax.numpy as jnp
from jax import lax
from jax.experimental import pallas as pl

COST_CLASS, COST_BBOX, COST_GIOU = 2.0, 5.0, 2.0
FOCAL_ALPHA = 0.25


def _matcher_body(logits_ref, pb_ref, lab_ref, tb_ref, idxi_ref, idxj_ref):
    _, nq, nc = logits_ref.shape
    ntgt = lab_ref.shape[2]

    logits = logits_ref[0]
    labels = lab_ref[0, 0, :]

    onehot = (labels[None, :] == lax.broadcasted_iota(jnp.int32, (nc, ntgt), 0)
              ).astype(jnp.float32)
    g = jnp.dot(logits, onehot, preferred_element_type=jnp.float32,
                precision=lax.Precision.HIGHEST)

    p = jax.nn.sigmoid(g)
    neg_cost = (1.0 - FOCAL_ALPHA) * (p * p) * (-jnp.log(1.0 - p + 1e-8))
    pos_cost = FOCAL_ALPHA * ((1.0 - p) * (1.0 - p)) * (-jnp.log(p + 1e-8))
    cost_class = pos_cost - neg_cost

    ocx = pb_ref[0, 0, :].reshape(nq, 1)
    ocy = pb_ref[0, 1, :].reshape(nq, 1)
    ow = pb_ref[0, 2, :].reshape(nq, 1)
    oh = pb_ref[0, 3, :].reshape(nq, 1)
    tcx = tb_ref[0, 0, :].reshape(1, ntgt)
    tcy = tb_ref[0, 1, :].reshape(1, ntgt)
    tw = tb_ref[0, 2, :].reshape(1, ntgt)
    th = tb_ref[0, 3, :].reshape(1, ntgt)

    cost_bbox = (jnp.abs(ocx - tcx) + jnp.abs(ocy - tcy)
                 + jnp.abs(ow - tw) + jnp.abs(oh - th))

    ox1, oy1 = ocx - 0.5 * ow, ocy - 0.5 * oh
    ox2, oy2 = ocx + 0.5 * ow, ocy + 0.5 * oh
    tx1, ty1 = tcx - 0.5 * tw, tcy - 0.5 * th
    tx2, ty2 = tcx + 0.5 * tw, tcy + 0.5 * th

    area1 = (ox2 - ox1) * (oy2 - oy1)
    area2 = (tx2 - tx1) * (ty2 - ty1)
    wx = jnp.maximum(jnp.minimum(ox2, tx2) - jnp.maximum(ox1, tx1), 0.0)
    wy = jnp.maximum(jnp.minimum(oy2, ty2) - jnp.maximum(oy1, ty1), 0.0)
    inter = wx * wy
    union = area1 + area2 - inter
    iou = inter / (union + 1e-9)
    w2x = jnp.maximum(jnp.maximum(ox2, tx2) - jnp.minimum(ox1, tx1), 0.0)
    w2y = jnp.maximum(jnp.maximum(oy2, ty2) - jnp.minimum(oy1, ty1), 0.0)
    area = w2x * w2y
    giou = iou - (area - union) / (area + 1e-9)

    C = COST_BBOX * cost_bbox + COST_CLASS * cost_class + COST_GIOU * (-giou)

    mn = jnp.min(C, axis=0)
    qio = lax.broadcasted_iota(jnp.int32, (nq, ntgt), 0)
    idx = jnp.min(jnp.where(C == mn[None, :], qio, nq), axis=0)
    idxi_ref[0] = idx.reshape(1, ntgt).astype(jnp.int32)
    idxj_ref[0] = lax.broadcasted_iota(jnp.int32, (1, ntgt), 1)


def kernel(pred_logits, pred_boxes, tgt_labels, tgt_boxes):
    bs, nq, nc = pred_logits.shape
    ntgt = tgt_labels.shape[1]
    pb = pred_boxes.transpose(0, 2, 1)
    tb = tgt_boxes.transpose(0, 2, 1)
    lab3 = tgt_labels.reshape(bs, 1, ntgt)

    idxi, idxj = pl.pallas_call(
        _matcher_body,
        grid=(bs,),
        in_specs=[
            pl.BlockSpec((1, nq, nc), lambda b: (b, 0, 0)),
            pl.BlockSpec((1, 4, nq), lambda b: (b, 0, 0)),
            pl.BlockSpec((1, 1, ntgt), lambda b: (b, 0, 0)),
            pl.BlockSpec((1, 4, ntgt), lambda b: (b, 0, 0)),
        ],
        out_specs=(
            pl.BlockSpec((1, 1, ntgt), lambda b: (b, 0, 0)),
            pl.BlockSpec((1, 1, ntgt), lambda b: (b, 0, 0)),
        ),
        out_shape=(
            jax.ShapeDtypeStruct((bs, 1, ntgt), jnp.int32),
            jax.ShapeDtypeStruct((bs, 1, ntgt), jnp.int32),
        ),
    )(pred_logits, pb, lab3, tb)
    return idxi.reshape(bs, ntgt), idxj.reshape(bs, ntgt)
